# baseline (speedup 1.0000x reference)
import functools

import jax
import jax.numpy as jnp
from jax import lax
from jax.experimental import pallas as pl
from jax.experimental.pallas import tpu as pltpu
from jax.experimental.pallas import tpu_sc as plsc

N_NODES = 10000
N_EDGES = 320000
D_EDGE = 4

NC = 2
NS = 16
E_PER_T = N_EDGES // NS
CHUNK_E = 2000
N_CHUNK = E_PER_T // CHUNK_E
CGROUPS = CHUNK_E // 16

DROWS = 80
DCOLS = 128
EROWS = 8
N_EUNITS = DROWS // EROWS

_mesh = plsc.VectorSubcoreMesh(
    core_axis_name="c", subcore_axis_name="s", num_cores=NC, num_subcores=NS
)


def _potts_body(attr_hbm, eu_hbm, ev_hbm, out_hbm,
                attr_a, attr_b, eu_a, eu_b, ev_a, ev_b,
                deg_v, ridx_v, esl_v, stage_v,
                acc_sh, acc_sm, sem_a, sem_u, sem_v):
    cid = lax.axis_index("c")
    sid = lax.axis_index("s")
    base = sid * E_PER_T
    attr_bufs = (attr_a, attr_b)
    eu_bufs = (eu_a, eu_b)
    ev_bufs = (ev_a, ev_b)

    def _start(k, buf):
        e0 = base + k * CHUNK_E
        ca = pltpu.async_copy(
            attr_hbm.at[pl.ds(e0 * D_EDGE, CHUNK_E * D_EDGE)], attr_bufs[buf], sem_a
        )
        cu = pltpu.async_copy(eu_hbm.at[pl.ds(e0, CHUNK_E)], eu_bufs[buf], sem_u)
        cv = pltpu.async_copy(ev_hbm.at[pl.ds(e0, CHUNK_E)], ev_bufs[buf], sem_v)
        return ca, cu, cv

    cps = _start(0, 0)

    zeros = jnp.zeros((16,), jnp.float32)

    def _zero_row(r, carry):
        for cc in range(DCOLS // 16):
            deg_v[r, pl.ds(cc * 16, 16)] = zeros
        return carry

    lax.fori_loop(0, DROWS, _zero_row, 0)

    iota16 = lax.iota(jnp.int32, 16)
    for i in range(DROWS // 16):
        ridx_v[pl.ds(i * 16, 16)] = iota16 + (i * 16)

    @pl.when(sid == 0)
    def _():
        pltpu.sync_copy(deg_v, acc_sh)
        acc_sm[0] = 0

    ones = jnp.ones((16,), jnp.float32)
    iota4 = iota16 * D_EDGE

    def _chunk_loop(buf, is0):
        attr_v = attr_bufs[buf]

        def _group(g, carry):
            f0 = iota4 + (g * 16 * D_EDGE)
            a0 = plsc.load_gather(attr_v, [f0])
            a1 = plsc.load_gather(attr_v, [f0 + 1])
            a2 = plsc.load_gather(attr_v, [f0 + 2])
            a3 = plsc.load_gather(attr_v, [f0 + 3])
            if is0:
                m = (a1 > a0) & (a1 >= a2) & (a1 >= a3)
            else:
                m = (a2 > a0) & (a2 > a1) & (a2 >= a3)
            e0 = g * 16
            u = eu_bufs[buf][pl.ds(e0, 16)]
            v = ev_bufs[buf][pl.ds(e0, 16)]
            ur = lax.shift_right_logical(u, 7)
            uc = lax.bitwise_and(u, 127)
            vr = lax.shift_right_logical(v, 7)
            vc = lax.bitwise_and(v, 127)
            plsc.addupdate_scatter(deg_v, [ur, uc], ones, mask=m)
            plsc.addupdate_scatter(deg_v, [vr, vc], ones, mask=m)
            return carry

        lax.fori_loop(0, CGROUPS, _group, 0)

    def _run(is0):
        copies = cps
        for k in range(N_CHUNK):
            buf = k % 2
            for cp in copies:
                cp.wait()
            if k + 1 < N_CHUNK:
                copies = _start(k + 1, 1 - buf)
            _chunk_loop(buf, is0)

    @pl.when(cid == 0)
    def _():
        _run(True)

    @pl.when(cid == 1)
    def _():
        _run(False)

    plsc.subcore_barrier()
    pltpu.sync_copy(deg_v, acc_sh.at[ridx_v], add=True)
    plsc.subcore_barrier()

    @pl.when(sid < N_EUNITS)
    def _():
        pltpu.sync_copy(acc_sh.at[pl.ds(sid * EROWS, EROWS)], esl_v)
        acc = jnp.zeros((16,), jnp.float32)
        for r in range(EROWS):
            for cc in range(DCOLS // 16):
                x = esl_v[r, pl.ds(cc * 16, 16)]
                acc = acc + x * x
        s = lax.reduce_sum_p.bind(acc, axes=(0,))
        plsc.fetch_and_add(acc_sm, s.astype(jnp.int32), subcore_id=0)

    plsc.subcore_barrier()

    @pl.when(sid == 0)
    def _():
        tot = acc_sm[0].astype(jnp.float32)
        stage_v[0, pl.ds(0, 16)] = jnp.full((16,), tot, jnp.float32)
        pltpu.sync_copy(stage_v.at[0], out_hbm.at[pl.ds(cid * 16, 16)])


@functools.partial(
    pl.kernel,
    out_type=jax.ShapeDtypeStruct((NC * 16,), jnp.float32),
    mesh=_mesh,
    scratch_types=[
        pltpu.VMEM((CHUNK_E * D_EDGE,), jnp.float32),
        pltpu.VMEM((CHUNK_E * D_EDGE,), jnp.float32),
        pltpu.VMEM((CHUNK_E,), jnp.int32),
        pltpu.VMEM((CHUNK_E,), jnp.int32),
        pltpu.VMEM((CHUNK_E,), jnp.int32),
        pltpu.VMEM((CHUNK_E,), jnp.int32),
        pltpu.VMEM((DROWS, DCOLS), jnp.float32),
        pltpu.VMEM((DROWS,), jnp.int32),
        pltpu.VMEM((EROWS, DCOLS), jnp.float32),
        pltpu.VMEM((1, 16), jnp.float32),
        pltpu.VMEM_SHARED((DROWS, DCOLS), jnp.float32),
        pltpu.SMEM((1,), jnp.int32),
        pltpu.SemaphoreType.DMA,
        pltpu.SemaphoreType.DMA,
        pltpu.SemaphoreType.DMA,
    ],
    compiler_params=pltpu.CompilerParams(needs_layout_passes=False),
)
def _potts_kernel(*args):
    _potts_body(*args)


def kernel(node_features, edge_attr, coupling_strength, edge_index):
    num_nodes = node_features.shape[0]
    out = _potts_kernel(edge_attr.reshape(-1), edge_index[0], edge_index[1])
    return (out[0] + out[16]) * coupling_strength / (2.0 * num_nodes)


# speedup vs baseline: 2.8967x; 2.8967x over previous
import functools

import jax
import jax.numpy as jnp
from jax import lax
from jax.experimental import pallas as pl
from jax.experimental.pallas import tpu as pltpu
from jax.experimental.pallas import tpu_sc as plsc

N_NODES = 10000
N_EDGES = 320000
D_EDGE = 4

NC = 2
NS = 16
NBLK = N_EDGES // 128
CBLK = 4
CHUNK_E = CBLK * 128
N_CHUNKS = NBLK // CBLK

DROWS = 80
DCOLS = 128
EROWS = 8
N_EUNITS = DROWS // EROWS

_mesh = plsc.VectorSubcoreMesh(
    core_axis_name="c", subcore_axis_name="s", num_cores=NC, num_subcores=NS
)


def _potts_body(attr_hbm, ei_hbm, out_hbm,
                attr_v, ei_v, deg_v, ridx_v, esl_v, stage_v,
                acc_sh, acc_sm, sem_a, sem_e):
    cid = lax.axis_index("c")
    sid = lax.axis_index("s")

    zeros = jnp.zeros((16,), jnp.float32)

    def _zero_row(r, carry):
        for cc in range(DCOLS // 16):
            deg_v[r, pl.ds(cc * 16, 16)] = zeros
        return carry

    lax.fori_loop(0, DROWS, _zero_row, 0)

    iota16 = lax.iota(jnp.int32, 16)
    for i in range(DROWS // 16):
        ridx_v[pl.ds(i * 16, 16)] = iota16 + (i * 16)

    @pl.when(sid == 0)
    def _():
        pltpu.sync_copy(deg_v, acc_sh)
        acc_sm[0] = 0

    ones = jnp.ones((16,), jnp.float32)
    n_iter = 39 + (sid == 0).astype(jnp.int32)

    def _make_chunk_loop(is0):
        def _chunk(i, carry):
            g = i * NS + sid
            ca = pltpu.async_copy(
                attr_hbm.at[pl.ds(g * (CHUNK_E * D_EDGE), CHUNK_E * D_EDGE)],
                attr_v, sem_a,
            )
            ce = pltpu.async_copy(
                ei_hbm.at[pl.ds(g * (CHUNK_E * 2), CHUNK_E * 2)], ei_v, sem_e
            )
            ca.wait()
            ce.wait()
            for gg in range(CHUNK_E // 16):
                bb, ss = gg >> 3, gg & 7
                ao = bb * 512 + ss * 16
                a0 = attr_v[pl.ds(ao, 16)]
                a1 = attr_v[pl.ds(ao + 128, 16)]
                a2 = attr_v[pl.ds(ao + 256, 16)]
                a3 = attr_v[pl.ds(ao + 384, 16)]
                if is0:
                    m = (a1 > a0) & (a1 >= a2) & (a1 >= a3)
                else:
                    m = (a2 > a0) & (a2 > a1) & (a2 >= a3)
                eo = bb * 256 + ss * 16
                u = ei_v[pl.ds(eo, 16)]
                v = ei_v[pl.ds(eo + 128, 16)]
                ur = lax.shift_right_logical(u, 7)
                uc = lax.bitwise_and(u, 127)
                vr = lax.shift_right_logical(v, 7)
                vc = lax.bitwise_and(v, 127)
                plsc.addupdate_scatter(deg_v, [ur, uc], ones, mask=m)
                plsc.addupdate_scatter(deg_v, [vr, vc], ones, mask=m)
            return carry

        return _chunk

    @pl.when(cid == 0)
    def _():
        lax.fori_loop(0, n_iter, _make_chunk_loop(True), 0)

    @pl.when(cid == 1)
    def _():
        lax.fori_loop(0, n_iter, _make_chunk_loop(False), 0)

    plsc.subcore_barrier()
    pltpu.sync_copy(deg_v, acc_sh.at[ridx_v], add=True)
    plsc.subcore_barrier()

    @pl.when(sid < N_EUNITS)
    def _():
        pltpu.sync_copy(acc_sh.at[pl.ds(sid * EROWS, EROWS)], esl_v)
        acc = jnp.zeros((16,), jnp.float32)
        for r in range(EROWS):
            for cc in range(DCOLS // 16):
                x = esl_v[r, pl.ds(cc * 16, 16)]
                acc = acc + x * x
        s = lax.reduce_sum_p.bind(acc, axes=(0,))
        plsc.fetch_and_add(acc_sm, s.astype(jnp.int32), subcore_id=0)

    plsc.subcore_barrier()

    @pl.when(sid == 0)
    def _():
        tot = acc_sm[0].astype(jnp.float32)
        stage_v[0, pl.ds(0, 16)] = jnp.full((16,), tot, jnp.float32)
        pltpu.sync_copy(stage_v.at[0], out_hbm.at[pl.ds(cid * 16, 16)])


@functools.partial(
    pl.kernel,
    out_type=jax.ShapeDtypeStruct((NC * 16,), jnp.float32),
    mesh=_mesh,
    scratch_types=[
        pltpu.VMEM((CHUNK_E * D_EDGE,), jnp.float32),
        pltpu.VMEM((CHUNK_E * 2,), jnp.int32),
        pltpu.VMEM((DROWS, DCOLS), jnp.float32),
        pltpu.VMEM((DROWS,), jnp.int32),
        pltpu.VMEM((EROWS, DCOLS), jnp.float32),
        pltpu.VMEM((1, 16), jnp.float32),
        pltpu.VMEM_SHARED((DROWS, DCOLS), jnp.float32),
        pltpu.SMEM((1,), jnp.int32),
        pltpu.SemaphoreType.DMA,
        pltpu.SemaphoreType.DMA,
    ],
    compiler_params=pltpu.CompilerParams(needs_layout_passes=False),
)
def _potts_kernel(*args):
    _potts_body(*args)


def kernel(node_features, edge_attr, coupling_strength, edge_index):
    num_nodes = node_features.shape[0]
    nblk = edge_attr.shape[0] // 128
    af = edge_attr.reshape(nblk, 128, D_EDGE).transpose(0, 2, 1).reshape(-1)
    eif = edge_index.reshape(2, nblk, 128).transpose(1, 0, 2).reshape(-1)
    out = _potts_kernel(af, eif)
    return (out[0] + out[16]) * coupling_strength / (2.0 * num_nodes)


# speedup vs baseline: 3.3270x; 1.1485x over previous
import functools

import jax
import jax.numpy as jnp
from jax import lax
from jax.experimental import pallas as pl
from jax.experimental.pallas import tpu as pltpu
from jax.experimental.pallas import tpu_sc as plsc

N_NODES = 10000
N_EDGES = 320000
D_EDGE = 4

NC = 2
NS = 16
NBLK = N_EDGES // 128
CBLK = 4
CHUNK_E = CBLK * 128
N_CHUNKS = NBLK // CBLK

DROWS = 80
DCOLS = 128
EROWS = 8
N_EUNITS = DROWS // EROWS

_mesh = plsc.VectorSubcoreMesh(
    core_axis_name="c", subcore_axis_name="s", num_cores=NC, num_subcores=NS
)


AW = CHUNK_E * D_EDGE
EW = CHUNK_E * 2
N_DBL = 20


def _potts_body(attr_hbm, ei_hbm, out_hbm,
                attr_a, attr_b, ei_a, ei_b, deg_v, ridx_v, esl_v, stage_v,
                acc_sh, acc_sm, sem_a0, sem_a1, sem_e0, sem_e1):
    cid = lax.axis_index("c")
    sid = lax.axis_index("s")

    zeros = jnp.zeros((16,), jnp.float32)

    def _zero_row(r, carry):
        for cc in range(DCOLS // 16):
            deg_v[r, pl.ds(cc * 16, 16)] = zeros
        return carry

    lax.fori_loop(0, DROWS, _zero_row, 0)

    iota16 = lax.iota(jnp.int32, 16)
    for i in range(DROWS // 16):
        ridx_v[pl.ds(i * 16, 16)] = iota16 + (i * 16)

    @pl.when(sid == 0)
    def _():
        pltpu.sync_copy(deg_v, acc_sh)
        acc_sm[0] = 0

    ones = jnp.ones((16,), jnp.float32)

    def _issue(i, av, ev_, sa, se):
        g = jnp.minimum(i * NS + sid, N_CHUNKS - 1)
        pltpu.async_copy(attr_hbm.at[pl.ds(g * AW, AW)], av, sa)
        pltpu.async_copy(ei_hbm.at[pl.ds(g * EW, EW)], ev_, se)

    def _wait(av, ev_, sa, se):
        pltpu.make_async_copy(attr_hbm.at[pl.ds(0, AW)], av, sa).wait()
        pltpu.make_async_copy(ei_hbm.at[pl.ds(0, EW)], ev_, se).wait()

    def _process(av, ev_, i, is0):
        @pl.when(i * NS + sid <= N_CHUNKS - 1)
        def _():
            for gg in range(CHUNK_E // 16):
                bb, ss = gg >> 3, gg & 7
                ao = bb * 512 + ss * 16
                a0 = av[pl.ds(ao, 16)]
                a1 = av[pl.ds(ao + 128, 16)]
                a2 = av[pl.ds(ao + 256, 16)]
                a3 = av[pl.ds(ao + 384, 16)]
                if is0:
                    m = (a1 > a0) & (a1 >= a2) & (a1 >= a3)
                else:
                    m = (a2 > a0) & (a2 > a1) & (a2 >= a3)
                eo = bb * 256 + ss * 16
                u = ev_[pl.ds(eo, 16)]
                v = ev_[pl.ds(eo + 128, 16)]
                ur = lax.shift_right_logical(u, 7)
                uc = lax.bitwise_and(u, 127)
                vr = lax.shift_right_logical(v, 7)
                vc = lax.bitwise_and(v, 127)
                plsc.addupdate_scatter(deg_v, [ur, uc], ones, mask=m)
                plsc.addupdate_scatter(deg_v, [vr, vc], ones, mask=m)

    def _make_dbl(is0):
        def _dbl(j, carry):
            i0 = 2 * j
            _wait(attr_a, ei_a, sem_a0, sem_e0)
            _issue(i0 + 1, attr_b, ei_b, sem_a1, sem_e1)
            _process(attr_a, ei_a, i0, is0)
            _wait(attr_b, ei_b, sem_a1, sem_e1)

            @pl.when(j < N_DBL - 1)
            def _():
                _issue(i0 + 2, attr_a, ei_a, sem_a0, sem_e0)

            _process(attr_b, ei_b, i0 + 1, is0)
            return carry

        return _dbl

    _issue(0, attr_a, ei_a, sem_a0, sem_e0)

    @pl.when(cid == 0)
    def _():
        lax.fori_loop(0, N_DBL, _make_dbl(True), 0)

    @pl.when(cid == 1)
    def _():
        lax.fori_loop(0, N_DBL, _make_dbl(False), 0)

    plsc.subcore_barrier()
    pltpu.sync_copy(deg_v, acc_sh.at[ridx_v], add=True)
    plsc.subcore_barrier()

    @pl.when(sid < N_EUNITS)
    def _():
        pltpu.sync_copy(acc_sh.at[pl.ds(sid * EROWS, EROWS)], esl_v)
        acc = jnp.zeros((16,), jnp.float32)
        for r in range(EROWS):
            for cc in range(DCOLS // 16):
                x = esl_v[r, pl.ds(cc * 16, 16)]
                acc = acc + x * x
        s = lax.reduce_sum_p.bind(acc, axes=(0,))
        plsc.fetch_and_add(acc_sm, s.astype(jnp.int32), subcore_id=0)

    plsc.subcore_barrier()

    @pl.when(sid == 0)
    def _():
        tot = acc_sm[0].astype(jnp.float32)
        stage_v[0, pl.ds(0, 16)] = jnp.full((16,), tot, jnp.float32)
        pltpu.sync_copy(stage_v.at[0], out_hbm.at[pl.ds(cid * 16, 16)])


@functools.partial(
    pl.kernel,
    out_type=jax.ShapeDtypeStruct((NC * 16,), jnp.float32),
    mesh=_mesh,
    scratch_types=[
        pltpu.VMEM((CHUNK_E * D_EDGE,), jnp.float32),
        pltpu.VMEM((CHUNK_E * D_EDGE,), jnp.float32),
        pltpu.VMEM((CHUNK_E * 2,), jnp.int32),
        pltpu.VMEM((CHUNK_E * 2,), jnp.int32),
        pltpu.VMEM((DROWS, DCOLS), jnp.float32),
        pltpu.VMEM((DROWS,), jnp.int32),
        pltpu.VMEM((EROWS, DCOLS), jnp.float32),
        pltpu.VMEM((1, 16), jnp.float32),
        pltpu.VMEM_SHARED((DROWS, DCOLS), jnp.float32),
        pltpu.SMEM((1,), jnp.int32),
        pltpu.SemaphoreType.DMA,
        pltpu.SemaphoreType.DMA,
        pltpu.SemaphoreType.DMA,
        pltpu.SemaphoreType.DMA,
    ],
    compiler_params=pltpu.CompilerParams(needs_layout_passes=False),
)
def _potts_kernel(*args):
    _potts_body(*args)


def kernel(node_features, edge_attr, coupling_strength, edge_index):
    num_nodes = node_features.shape[0]
    nblk = edge_attr.shape[0] // 128
    af = edge_attr.reshape(nblk, 128, D_EDGE).transpose(0, 2, 1).reshape(-1)
    eif = edge_index.reshape(2, nblk, 128).transpose(1, 0, 2).reshape(-1)
    out = _potts_kernel(af, eif)
    return (out[0] + out[16]) * coupling_strength / (2.0 * num_nodes)


# speedup vs baseline: 3.3408x; 1.0041x over previous
import functools

import jax
import jax.numpy as jnp
from jax import lax
from jax.experimental import pallas as pl
from jax.experimental.pallas import tpu as pltpu
from jax.experimental.pallas import tpu_sc as plsc

N_NODES = 10000
N_EDGES = 320000
D_EDGE = 4

NC = 2
NS = 16
NBLK = N_EDGES // 128
CBLK = 4
CHUNK_E = CBLK * 128
N_CHUNKS = NBLK // CBLK

DROWS = 80
DCOLS = 128
EROWS = 8
N_EUNITS = DROWS // EROWS

_mesh = plsc.VectorSubcoreMesh(
    core_axis_name="c", subcore_axis_name="s", num_cores=NC, num_subcores=NS
)


AW = CHUNK_E * D_EDGE
EW = CHUNK_E * 2
N_DBL = 20


def _potts_body(attr_hbm, ei_hbm, out_hbm,
                attr_a, attr_b, ei_a, ei_b, deg_v, ridx_v, esl_v, stage_v,
                acc_sh, acc_sm, sem_a0, sem_a1, sem_e0, sem_e1):
    cid = lax.axis_index("c")
    sid = lax.axis_index("s")

    zeros = jnp.zeros((16,), jnp.float32)

    def _zero_row(r, carry):
        for cc in range(DCOLS // 16):
            deg_v[r, pl.ds(cc * 16, 16)] = zeros
        return carry

    lax.fori_loop(0, DROWS, _zero_row, 0)

    iota16 = lax.iota(jnp.int32, 16)
    for i in range(DROWS // 16):
        ridx_v[pl.ds(i * 16, 16)] = iota16 + (i * 16)

    @pl.when(sid == 0)
    def _():
        pltpu.sync_copy(deg_v, acc_sh)
        acc_sm[0] = 0

    ones = jnp.ones((16,), jnp.float32)
    is0v = jnp.full((16,), cid == 0)

    def _issue(i, av, ev_, sa, se):
        g = jnp.minimum(i * NS + sid, N_CHUNKS - 1)
        pltpu.async_copy(attr_hbm.at[pl.ds(g * AW, AW)], av, sa)
        pltpu.async_copy(ei_hbm.at[pl.ds(g * EW, EW)], ev_, se)

    def _wait(av, ev_, sa, se):
        pltpu.make_async_copy(attr_hbm.at[pl.ds(0, AW)], av, sa).wait()
        pltpu.make_async_copy(ei_hbm.at[pl.ds(0, EW)], ev_, se).wait()

    def _process(av, ev_, i):
        @pl.when(i * NS + sid <= N_CHUNKS - 1)
        def _():
            for gg in range(CHUNK_E // 16):
                bb, ss = gg >> 3, gg & 7
                ao = bb * 512 + ss * 16
                a0 = av[pl.ds(ao, 16)]
                a1 = av[pl.ds(ao + 128, 16)]
                a2 = av[pl.ds(ao + 256, 16)]
                a3 = av[pl.ds(ao + 384, 16)]
                p = jnp.where(is0v, a1, a2)
                tie = jnp.where(is0v, a1 >= a2, a2 > a1)
                m = (p > a0) & (p >= a3) & tie
                eo = bb * 256 + ss * 16
                u = ev_[pl.ds(eo, 16)]
                v = ev_[pl.ds(eo + 128, 16)]
                ur = lax.shift_right_logical(u, 7)
                uc = lax.bitwise_and(u, 127)
                vr = lax.shift_right_logical(v, 7)
                vc = lax.bitwise_and(v, 127)
                plsc.addupdate_scatter(deg_v, [ur, uc], ones, mask=m)
                plsc.addupdate_scatter(deg_v, [vr, vc], ones, mask=m)

    def _dbl(j, carry):
        i0 = 2 * j
        _wait(attr_a, ei_a, sem_a0, sem_e0)
        _issue(i0 + 1, attr_b, ei_b, sem_a1, sem_e1)
        _process(attr_a, ei_a, i0)
        _wait(attr_b, ei_b, sem_a1, sem_e1)

        @pl.when(j < N_DBL - 1)
        def _():
            _issue(i0 + 2, attr_a, ei_a, sem_a0, sem_e0)

        _process(attr_b, ei_b, i0 + 1)
        return carry

    _issue(0, attr_a, ei_a, sem_a0, sem_e0)
    lax.fori_loop(0, N_DBL, _dbl, 0)

    plsc.subcore_barrier()
    pltpu.sync_copy(deg_v, acc_sh.at[ridx_v], add=True)
    plsc.subcore_barrier()

    @pl.when(sid < N_EUNITS)
    def _():
        pltpu.sync_copy(acc_sh.at[pl.ds(sid * EROWS, EROWS)], esl_v)
        acc = jnp.zeros((16,), jnp.float32)
        for r in range(EROWS):
            for cc in range(DCOLS // 16):
                x = esl_v[r, pl.ds(cc * 16, 16)]
                acc = acc + x * x
        s = lax.reduce_sum_p.bind(acc, axes=(0,))
        plsc.fetch_and_add(acc_sm, s.astype(jnp.int32), subcore_id=0)

    plsc.subcore_barrier()

    @pl.when(sid == 0)
    def _():
        tot = acc_sm[0].astype(jnp.float32)
        stage_v[0, pl.ds(0, 16)] = jnp.full((16,), tot, jnp.float32)
        pltpu.sync_copy(stage_v.at[0], out_hbm.at[pl.ds(cid * 16, 16)])


@functools.partial(
    pl.kernel,
    out_type=jax.ShapeDtypeStruct((NC * 16,), jnp.float32),
    mesh=_mesh,
    scratch_types=[
        pltpu.VMEM((CHUNK_E * D_EDGE,), jnp.float32),
        pltpu.VMEM((CHUNK_E * D_EDGE,), jnp.float32),
        pltpu.VMEM((CHUNK_E * 2,), jnp.int32),
        pltpu.VMEM((CHUNK_E * 2,), jnp.int32),
        pltpu.VMEM((DROWS, DCOLS), jnp.float32),
        pltpu.VMEM((DROWS,), jnp.int32),
        pltpu.VMEM((EROWS, DCOLS), jnp.float32),
        pltpu.VMEM((1, 16), jnp.float32),
        pltpu.VMEM_SHARED((DROWS, DCOLS), jnp.float32),
        pltpu.SMEM((1,), jnp.int32),
        pltpu.SemaphoreType.DMA,
        pltpu.SemaphoreType.DMA,
        pltpu.SemaphoreType.DMA,
        pltpu.SemaphoreType.DMA,
    ],
    compiler_params=pltpu.CompilerParams(needs_layout_passes=False),
)
def _potts_kernel(*args):
    _potts_body(*args)


def kernel(node_features, edge_attr, coupling_strength, edge_index):
    num_nodes = node_features.shape[0]
    nblk = edge_attr.shape[0] // 128
    af = edge_attr.reshape(nblk, 128, D_EDGE).transpose(0, 2, 1).reshape(-1)
    eif = edge_index.reshape(2, nblk, 128).transpose(1, 0, 2).reshape(-1)
    out = _potts_kernel(af, eif)
    return (out[0] + out[16]) * coupling_strength / (2.0 * num_nodes)


# speedup vs baseline: 3.8642x; 1.1567x over previous
import functools

import jax
import jax.numpy as jnp
from jax import lax
from jax.experimental import pallas as pl
from jax.experimental.pallas import tpu as pltpu
from jax.experimental.pallas import tpu_sc as plsc

N_NODES = 10000
N_EDGES = 320000
D_EDGE = 4

NC = 2
NS = 16
NW = NC * NS
NBLK = N_EDGES // 128
CBLK = 4
CHUNK_E = CBLK * 128
N_CHUNKS = NBLK // CBLK

DROWS = 80
DCOLS = 128

AW = CHUNK_E * D_EDGE
EW = CHUNK_E * 2
N_DBL = 10

_mesh = plsc.VectorSubcoreMesh(
    core_axis_name="c", subcore_axis_name="s", num_cores=NC, num_subcores=NS
)


def _degrees_body(attr_hbm, ei_hbm, out_hbm,
                  attr_a, attr_b, ei_a, ei_b, deg1_v, deg2_v, ridx_v,
                  acc1_sh, acc2_sh, sem_a0, sem_a1, sem_e0, sem_e1):
    cid = lax.axis_index("c")
    sid = lax.axis_index("s")
    wid = sid * NC + cid

    def _issue(i, av, ev_, sa, se):
        g = jnp.minimum(i * NW + wid, N_CHUNKS - 1)
        pltpu.async_copy(attr_hbm.at[pl.ds(g * AW, AW)], av, sa)
        pltpu.async_copy(ei_hbm.at[pl.ds(g * EW, EW)], ev_, se)

    def _wait(av, ev_, sa, se):
        pltpu.make_async_copy(attr_hbm.at[pl.ds(0, AW)], av, sa).wait()
        pltpu.make_async_copy(ei_hbm.at[pl.ds(0, EW)], ev_, se).wait()

    _issue(0, attr_a, ei_a, sem_a0, sem_e0)

    zeros = jnp.zeros((16,), jnp.float32)

    def _zero_row(r, carry):
        for cc in range(DCOLS // 16):
            deg1_v[r, pl.ds(cc * 16, 16)] = zeros
            deg2_v[r, pl.ds(cc * 16, 16)] = zeros
        return carry

    lax.fori_loop(0, DROWS, _zero_row, 0)

    iota16 = lax.iota(jnp.int32, 16)
    for i in range(DROWS // 16):
        ridx_v[pl.ds(i * 16, 16)] = iota16 + (i * 16)

    @pl.when(sid == 0)
    def _():
        pltpu.sync_copy(deg1_v, acc1_sh)
        pltpu.sync_copy(deg2_v, acc2_sh)

    ones = jnp.ones((16,), jnp.float32)

    def _process(av, ev_, i):
        @pl.when(i * NW + wid <= N_CHUNKS - 1)
        def _():
            for gg in range(CHUNK_E // 16):
                bb, ss = gg >> 3, gg & 7
                ao = bb * 512 + ss * 16
                a0 = av[pl.ds(ao, 16)]
                a1 = av[pl.ds(ao + 128, 16)]
                a2 = av[pl.ds(ao + 256, 16)]
                a3 = av[pl.ds(ao + 384, 16)]
                m1 = (a1 > a0) & (a1 >= a2) & (a1 >= a3)
                m2 = (a2 > a0) & (a2 > a1) & (a2 >= a3)
                eo = bb * 256 + ss * 16
                u = ev_[pl.ds(eo, 16)]
                v = ev_[pl.ds(eo + 128, 16)]
                ur = lax.shift_right_logical(u, 7)
                uc = lax.bitwise_and(u, 127)
                vr = lax.shift_right_logical(v, 7)
                vc = lax.bitwise_and(v, 127)
                plsc.addupdate_scatter(deg1_v, [ur, uc], ones, mask=m1)
                plsc.addupdate_scatter(deg1_v, [vr, vc], ones, mask=m1)
                plsc.addupdate_scatter(deg2_v, [ur, uc], ones, mask=m2)
                plsc.addupdate_scatter(deg2_v, [vr, vc], ones, mask=m2)

    def _dbl(j, carry):
        i0 = 2 * j
        _wait(attr_a, ei_a, sem_a0, sem_e0)
        _issue(i0 + 1, attr_b, ei_b, sem_a1, sem_e1)
        _process(attr_a, ei_a, i0)
        _wait(attr_b, ei_b, sem_a1, sem_e1)

        @pl.when(j < N_DBL - 1)
        def _():
            _issue(i0 + 2, attr_a, ei_a, sem_a0, sem_e0)

        _process(attr_b, ei_b, i0 + 1)
        return carry

    lax.fori_loop(0, N_DBL, _dbl, 0)

    plsc.subcore_barrier()
    pltpu.sync_copy(deg1_v, acc1_sh.at[ridx_v], add=True)
    pltpu.sync_copy(deg2_v, acc2_sh.at[ridx_v], add=True)
    plsc.subcore_barrier()

    @pl.when(sid == 0)
    def _():
        pltpu.sync_copy(acc1_sh, out_hbm.at[cid, 0])
        pltpu.sync_copy(acc2_sh, out_hbm.at[cid, 1])


@functools.partial(
    pl.kernel,
    out_type=jax.ShapeDtypeStruct((NC, 2, DROWS, DCOLS), jnp.float32),
    mesh=_mesh,
    scratch_types=[
        pltpu.VMEM((AW,), jnp.float32),
        pltpu.VMEM((AW,), jnp.float32),
        pltpu.VMEM((EW,), jnp.int32),
        pltpu.VMEM((EW,), jnp.int32),
        pltpu.VMEM((DROWS, DCOLS), jnp.float32),
        pltpu.VMEM((DROWS, DCOLS), jnp.float32),
        pltpu.VMEM((DROWS,), jnp.int32),
        pltpu.VMEM_SHARED((DROWS, DCOLS), jnp.float32),
        pltpu.VMEM_SHARED((DROWS, DCOLS), jnp.float32),
        pltpu.SemaphoreType.DMA,
        pltpu.SemaphoreType.DMA,
        pltpu.SemaphoreType.DMA,
        pltpu.SemaphoreType.DMA,
    ],
    compiler_params=pltpu.CompilerParams(needs_layout_passes=False),
)
def _degrees_kernel(*args):
    _degrees_body(*args)


HALF = 2 * DROWS * DCOLS
CHUNK = HALF // NS
CGROUPS = CHUNK // 16


def _energy_body(part_hbm, out_hbm, p0_v, p1_v, stage_v, acc_sm, sem0, sem1):
    cid = lax.axis_index("c")
    sid = lax.axis_index("s")

    @pl.when((cid == 0) & (sid == 0))
    def _():
        acc_sm[0] = 0

    plsc.subcore_barrier()

    @pl.when(cid == 0)
    def _():
        o = sid * CHUNK
        c0 = pltpu.async_copy(part_hbm.at[pl.ds(o, CHUNK)], p0_v, sem0)
        c1 = pltpu.async_copy(part_hbm.at[pl.ds(HALF + o, CHUNK)], p1_v, sem1)
        c0.wait()
        c1.wait()
        acc = jnp.zeros((16,), jnp.float32)
        for g in range(CGROUPS):
            x = p0_v[pl.ds(g * 16, 16)] + p1_v[pl.ds(g * 16, 16)]
            acc = acc + x * x
        s = lax.reduce_sum_p.bind(acc, axes=(0,))
        plsc.fetch_and_add(acc_sm, s.astype(jnp.int32), subcore_id=0)

    plsc.subcore_barrier()

    @pl.when((cid == 0) & (sid == 0))
    def _():
        tot = acc_sm[0].astype(jnp.float32)
        stage_v[0, pl.ds(0, 16)] = jnp.full((16,), tot, jnp.float32)
        pltpu.sync_copy(stage_v.at[0], out_hbm)


@functools.partial(
    pl.kernel,
    out_type=jax.ShapeDtypeStruct((16,), jnp.float32),
    mesh=_mesh,
    scratch_types=[
        pltpu.VMEM((CHUNK,), jnp.float32),
        pltpu.VMEM((CHUNK,), jnp.float32),
        pltpu.VMEM((1, 16), jnp.float32),
        pltpu.SMEM((1,), jnp.int32),
        pltpu.SemaphoreType.DMA,
        pltpu.SemaphoreType.DMA,
    ],
    compiler_params=pltpu.CompilerParams(needs_layout_passes=False),
)
def _energy_kernel(*args):
    _energy_body(*args)


def kernel(node_features, edge_attr, coupling_strength, edge_index):
    num_nodes = node_features.shape[0]
    nblk = edge_attr.shape[0] // 128
    af = edge_attr.reshape(nblk, 128, D_EDGE).transpose(0, 2, 1).reshape(-1)
    eif = edge_index.reshape(2, nblk, 128).transpose(1, 0, 2).reshape(-1)
    part = _degrees_kernel(af, eif)
    esum = _energy_kernel(part.reshape(-1))
    return esum[0] * coupling_strength / (2.0 * num_nodes)
